# baseline (speedup 1.0000x reference)
import functools

import jax
import jax.numpy as jnp
from jax import lax
from jax.experimental import pallas as pl
from jax.experimental.pallas import tpu as pltpu
from jax.experimental.pallas import tpu_sc as plsc

N_V = 10000
N_E = 5000
NNZ = 320000
D = 128

NC = 2
NS = 16
NW = NC * NS
K = 128
C = 100
C1 = 58
NNZ_PAD = NS * (C + C1) * K
E_ACC = 5120
V_ACC = 10240


def _sc_segment_sum(table, gidx, sidx, acc_rows):
    mesh = plsc.VectorSubcoreMesh(core_axis_name="c", subcore_axis_name="s")
    rows_per_tile = acc_rows // NS

    @functools.partial(
        pl.kernel,
        mesh=mesh,
        out_type=jax.ShapeDtypeStruct((NC, acc_rows, D), jnp.float32),
        scratch_types=[
            pltpu.VMEM((C, K), jnp.int32),
            pltpu.VMEM((C, K), jnp.int32),
            pltpu.VMEM((K, D), jnp.float32),
            pltpu.VMEM((32, D), jnp.float32),
            pltpu.VMEM_SHARED((acc_rows, D), jnp.float32),
            pltpu.SemaphoreType.DMA,
        ],
    )
    def body(table_h, gidx_h, sidx_h, out_h, gidx_v, sidx_v, rows_v, zbuf,
             acc, sem):
        c = lax.axis_index("c")
        s = lax.axis_index("s")
        wid = s * NC + c
        start = s * rows_per_tile

        def zrow(i, _):
            def zlane(j, _):
                zbuf[i, pl.ds(j * 16, 16)] = jnp.zeros((16,), jnp.float32)
                return 0
            lax.fori_loop(0, D // 16, zlane, 0)
            return 0
        lax.fori_loop(0, 32, zrow, 0)

        def zcopy(i, _):
            pltpu.sync_copy(zbuf, acc.at[pl.ds(start + i * 32, 32)])
            return 0
        lax.fori_loop(0, rows_per_tile // 32, zcopy, 0)

        plsc.subcore_barrier()

        pltpu.sync_copy(gidx_h.at[wid], gidx_v)
        pltpu.sync_copy(sidx_h.at[wid], sidx_v)

        def chunk(j, _):
            pltpu.async_copy(table_h.at[gidx_v.at[j]], rows_v, sem).wait()
            pltpu.sync_copy(rows_v, acc.at[sidx_v.at[j]], add=True)
            return 0

        @pl.when(c == 1)
        def _big_share():
            lax.fori_loop(0, C, chunk, 0)

        @pl.when(c == 0)
        def _small_share():
            lax.fori_loop(0, C1, chunk, 0)

        plsc.subcore_barrier()

        pltpu.sync_copy(acc.at[pl.ds(start, rows_per_tile)],
                        out_h.at[c, pl.ds(start, rows_per_tile)])

    return body(table, gidx, sidx)


def _tc_matmul(X, wT):
    n = X.shape[0]
    blk = 1000

    def mm(x_ref, w_ref, o_ref):
        o_ref[...] = jnp.dot(x_ref[...], w_ref[...],
                             preferred_element_type=jnp.float32)

    return pl.pallas_call(
        mm,
        grid=(n // blk,),
        in_specs=[
            pl.BlockSpec((blk, D), lambda i: (i, 0)),
            pl.BlockSpec((D, D), lambda i: (0, 0)),
        ],
        out_specs=pl.BlockSpec((blk, D), lambda i: (i, 0)),
        out_shape=jax.ShapeDtypeStruct((n, D), jnp.float32),
    )(X, wT)


def _tc_combine_scale(p0, p1, scales):
    n = p0.shape[0]
    blk = 1000

    def f(a_ref, b_ref, *rest):
        s_refs, o_ref = rest[:-1], rest[-1]
        acc = a_ref[...] + b_ref[...]
        for s_ref in s_refs:
            acc = acc * s_ref[...]
        o_ref[...] = acc

    return pl.pallas_call(
        f,
        grid=(n // blk,),
        in_specs=[pl.BlockSpec((blk, D), lambda i: (i, 0)),
                  pl.BlockSpec((blk, D), lambda i: (i, 0))] +
                 [pl.BlockSpec((blk, 1), lambda i: (i, 0))] * len(scales),
        out_specs=pl.BlockSpec((blk, D), lambda i: (i, 0)),
        out_shape=jax.ShapeDtypeStruct((n, D), jnp.float32),
    )(p0, p1, *scales)


def _make_idx(src, fill):
    pad = NNZ_PAD - NNZ
    a = jnp.concatenate([src, jnp.full((pad,), fill, jnp.int32)])
    nfast = NS * C * K
    fast = a[:nfast].reshape(NS, C, K)
    slow = jnp.pad(a[nfast:].reshape(NS, C1, K),
                   ((0, 0), (0, C - C1), (0, 0)), constant_values=fill)
    return jnp.stack([slow, fast], axis=1).reshape(NW, C, K)


@jax.jit
def kernel(X, vertex, edges, degE, degV, W_edge, lin_w):
    g1 = _make_idx(vertex, 0)
    s1 = _make_idx(edges, N_E)
    g2 = _make_idx(edges, 0)
    s2 = _make_idx(vertex, N_V)

    Xl = _tc_matmul(X, lin_w.T)
    pe = _sc_segment_sum(Xl, g1, s1, E_ACC)
    Xe = _tc_combine_scale(pe[0, :N_E], pe[1, :N_E], [degE, W_edge])
    pv = _sc_segment_sum(Xe, g2, s2, V_ACC)
    Xv = _tc_combine_scale(pv[0, :N_V], pv[1, :N_V], [degV])
    return Xv


# speedup vs baseline: 1.0530x; 1.0530x over previous
import functools

import jax
import jax.numpy as jnp
from jax import lax
from jax.experimental import pallas as pl
from jax.experimental.pallas import tpu as pltpu
from jax.experimental.pallas import tpu_sc as plsc

N_V = 10000
N_E = 5000
NNZ = 320000
D = 128

NC = 2
NS = 16
NW = NC * NS
K = 128
C = 108
C1 = 50
NNZ_PAD = NS * (C + C1) * K
E_ACC = 5120
V_ACC = 10240


def _sc_segment_sum(table, gidx, sidx, acc_rows):
    mesh = plsc.VectorSubcoreMesh(core_axis_name="c", subcore_axis_name="s")
    rows_per_tile = acc_rows // NS

    @functools.partial(
        pl.kernel,
        mesh=mesh,
        out_type=jax.ShapeDtypeStruct((NC, acc_rows, D), jnp.float32),
        scratch_types=[
            pltpu.VMEM((C, K), jnp.int32),
            pltpu.VMEM((C, K), jnp.int32),
            pltpu.VMEM((K, D), jnp.float32),
            pltpu.VMEM((32, D), jnp.float32),
            pltpu.VMEM_SHARED((acc_rows, D), jnp.float32),
            pltpu.SemaphoreType.DMA,
        ],
    )
    def body(table_h, gidx_h, sidx_h, out_h, gidx_v, sidx_v, rows_v, zbuf,
             acc, sem):
        c = lax.axis_index("c")
        s = lax.axis_index("s")
        wid = s * NC + c
        start = s * rows_per_tile

        def zrow(i, _):
            def zlane(j, _):
                zbuf[i, pl.ds(j * 16, 16)] = jnp.zeros((16,), jnp.float32)
                return 0
            lax.fori_loop(0, D // 16, zlane, 0)
            return 0
        lax.fori_loop(0, 32, zrow, 0)

        def zcopy(i, _):
            pltpu.sync_copy(zbuf, acc.at[pl.ds(start + i * 32, 32)])
            return 0
        lax.fori_loop(0, rows_per_tile // 32, zcopy, 0)

        plsc.subcore_barrier()

        pltpu.sync_copy(gidx_h.at[wid], gidx_v)
        pltpu.sync_copy(sidx_h.at[wid], sidx_v)

        def chunk(j, _):
            pltpu.async_copy(table_h.at[gidx_v.at[j]], rows_v, sem).wait()
            pltpu.sync_copy(rows_v, acc.at[sidx_v.at[j]], add=True)
            return 0

        @pl.when(c == 1)
        def _big_share():
            lax.fori_loop(0, C, chunk, 0)

        @pl.when(c == 0)
        def _small_share():
            lax.fori_loop(0, C1, chunk, 0)

        plsc.subcore_barrier()

        pltpu.sync_copy(acc.at[pl.ds(start, rows_per_tile)],
                        out_h.at[c, pl.ds(start, rows_per_tile)])

    return body(table, gidx, sidx)


def _tc_matmul(X, wT):
    n = X.shape[0]
    blk = 1000

    def mm(x_ref, w_ref, o_ref):
        o_ref[...] = jnp.dot(x_ref[...], w_ref[...],
                             preferred_element_type=jnp.float32)

    return pl.pallas_call(
        mm,
        grid=(n // blk,),
        in_specs=[
            pl.BlockSpec((blk, D), lambda i: (i, 0)),
            pl.BlockSpec((D, D), lambda i: (0, 0)),
        ],
        out_specs=pl.BlockSpec((blk, D), lambda i: (i, 0)),
        out_shape=jax.ShapeDtypeStruct((n, D), jnp.float32),
    )(X, wT)


def _tc_combine_scale(p0, p1, scales):
    n = p0.shape[0]
    blk = 1000

    def f(a_ref, b_ref, *rest):
        s_refs, o_ref = rest[:-1], rest[-1]
        acc = a_ref[...] + b_ref[...]
        for s_ref in s_refs:
            acc = acc * s_ref[...]
        o_ref[...] = acc

    return pl.pallas_call(
        f,
        grid=(n // blk,),
        in_specs=[pl.BlockSpec((blk, D), lambda i: (i, 0)),
                  pl.BlockSpec((blk, D), lambda i: (i, 0))] +
                 [pl.BlockSpec((blk, 1), lambda i: (i, 0))] * len(scales),
        out_specs=pl.BlockSpec((blk, D), lambda i: (i, 0)),
        out_shape=jax.ShapeDtypeStruct((n, D), jnp.float32),
    )(p0, p1, *scales)


def _make_idx(src, fill):
    pad = NNZ_PAD - NNZ
    a = jnp.concatenate([src, jnp.full((pad,), fill, jnp.int32)])
    nfast = NS * C * K
    fast = a[:nfast].reshape(NS, C, K)
    slow = jnp.pad(a[nfast:].reshape(NS, C1, K),
                   ((0, 0), (0, C - C1), (0, 0)), constant_values=fill)
    return jnp.stack([slow, fast], axis=1).reshape(NW, C, K)


@jax.jit
def kernel(X, vertex, edges, degE, degV, W_edge, lin_w):
    g1 = _make_idx(vertex, 0)
    s1 = _make_idx(edges, N_E)
    g2 = _make_idx(edges, 0)
    s2 = _make_idx(vertex, N_V)

    Xl = _tc_matmul(X, lin_w.T)
    pe = _sc_segment_sum(Xl, g1, s1, E_ACC)
    Xe = _tc_combine_scale(pe[0, :N_E], pe[1, :N_E], [degE, W_edge])
    pv = _sc_segment_sum(Xe, g2, s2, V_ACC)
    Xv = _tc_combine_scale(pv[0, :N_V], pv[1, :N_V], [degV])
    return Xv


# speedup vs baseline: 1.0636x; 1.0101x over previous
import functools

import jax
import jax.numpy as jnp
from jax import lax
from jax.experimental import pallas as pl
from jax.experimental.pallas import tpu as pltpu
from jax.experimental.pallas import tpu_sc as plsc

N_V = 10000
N_E = 5000
NNZ = 320000
D = 128

NC = 2
NS = 16
NW = NC * NS
K = 128
C = 112
C1 = 46
NNZ_PAD = NS * (C + C1) * K
E_ACC = 5120
V_ACC = 10240


def _sc_segment_sum(table, gidx, sidx, acc_rows):
    mesh = plsc.VectorSubcoreMesh(core_axis_name="c", subcore_axis_name="s")
    rows_per_tile = acc_rows // NS

    @functools.partial(
        pl.kernel,
        mesh=mesh,
        out_type=jax.ShapeDtypeStruct((NC, acc_rows, D), jnp.float32),
        scratch_types=[
            pltpu.VMEM((C, K), jnp.int32),
            pltpu.VMEM((C, K), jnp.int32),
            pltpu.VMEM((K, D), jnp.float32),
            pltpu.VMEM((32, D), jnp.float32),
            pltpu.VMEM_SHARED((acc_rows, D), jnp.float32),
            pltpu.SemaphoreType.DMA,
        ],
    )
    def body(table_h, gidx_h, sidx_h, out_h, gidx_v, sidx_v, rows_v, zbuf,
             acc, sem):
        c = lax.axis_index("c")
        s = lax.axis_index("s")
        wid = s * NC + c
        start = s * rows_per_tile

        def zrow(i, _):
            def zlane(j, _):
                zbuf[i, pl.ds(j * 16, 16)] = jnp.zeros((16,), jnp.float32)
                return 0
            lax.fori_loop(0, D // 16, zlane, 0)
            return 0
        lax.fori_loop(0, 32, zrow, 0)

        def zcopy(i, _):
            pltpu.sync_copy(zbuf, acc.at[pl.ds(start + i * 32, 32)])
            return 0
        lax.fori_loop(0, rows_per_tile // 32, zcopy, 0)

        plsc.subcore_barrier()

        pltpu.sync_copy(gidx_h.at[wid], gidx_v)
        pltpu.sync_copy(sidx_h.at[wid], sidx_v)

        def chunk(j, _):
            pltpu.async_copy(table_h.at[gidx_v.at[j]], rows_v, sem).wait()
            pltpu.sync_copy(rows_v, acc.at[sidx_v.at[j]], add=True)
            return 0

        @pl.when(c == 1)
        def _big_share():
            lax.fori_loop(0, C, chunk, 0)

        @pl.when(c == 0)
        def _small_share():
            lax.fori_loop(0, C1, chunk, 0)

        plsc.subcore_barrier()

        pltpu.sync_copy(acc.at[pl.ds(start, rows_per_tile)],
                        out_h.at[c, pl.ds(start, rows_per_tile)])

    return body(table, gidx, sidx)


def _tc_matmul(X, wT):
    n = X.shape[0]
    blk = 1000

    def mm(x_ref, w_ref, o_ref):
        o_ref[...] = jnp.dot(x_ref[...], w_ref[...],
                             preferred_element_type=jnp.float32)

    return pl.pallas_call(
        mm,
        grid=(n // blk,),
        in_specs=[
            pl.BlockSpec((blk, D), lambda i: (i, 0)),
            pl.BlockSpec((D, D), lambda i: (0, 0)),
        ],
        out_specs=pl.BlockSpec((blk, D), lambda i: (i, 0)),
        out_shape=jax.ShapeDtypeStruct((n, D), jnp.float32),
    )(X, wT)


def _tc_combine_scale(p0, p1, scales):
    n = p0.shape[0]
    blk = 1000

    def f(a_ref, b_ref, *rest):
        s_refs, o_ref = rest[:-1], rest[-1]
        acc = a_ref[...] + b_ref[...]
        for s_ref in s_refs:
            acc = acc * s_ref[...]
        o_ref[...] = acc

    return pl.pallas_call(
        f,
        grid=(n // blk,),
        in_specs=[pl.BlockSpec((blk, D), lambda i: (i, 0)),
                  pl.BlockSpec((blk, D), lambda i: (i, 0))] +
                 [pl.BlockSpec((blk, 1), lambda i: (i, 0))] * len(scales),
        out_specs=pl.BlockSpec((blk, D), lambda i: (i, 0)),
        out_shape=jax.ShapeDtypeStruct((n, D), jnp.float32),
    )(p0, p1, *scales)


def _make_idx(src, fill):
    pad = NNZ_PAD - NNZ
    a = jnp.concatenate([src, jnp.full((pad,), fill, jnp.int32)])
    nfast = NS * C * K
    fast = a[:nfast].reshape(NS, C, K)
    slow = jnp.pad(a[nfast:].reshape(NS, C1, K),
                   ((0, 0), (0, C - C1), (0, 0)), constant_values=fill)
    return jnp.stack([slow, fast], axis=1).reshape(NW, C, K)


@jax.jit
def kernel(X, vertex, edges, degE, degV, W_edge, lin_w):
    g1 = _make_idx(vertex, 0)
    s1 = _make_idx(edges, N_E)
    g2 = _make_idx(edges, 0)
    s2 = _make_idx(vertex, N_V)

    Xl = _tc_matmul(X, lin_w.T)
    pe = _sc_segment_sum(Xl, g1, s1, E_ACC)
    Xe = _tc_combine_scale(pe[0, :N_E], pe[1, :N_E], [degE, W_edge])
    pv = _sc_segment_sum(Xe, g2, s2, V_ACC)
    Xv = _tc_combine_scale(pv[0, :N_V], pv[1, :N_V], [degV])
    return Xv
